# baseline (speedup 1.0000x reference)
import jax
import jax.numpy as jnp
from jax.experimental import pallas as pl

N = 10000
E = 320000


def _cls_kernel(h_ref, w_ref, b_ref, o_ref):
    o_ref[...] = h_ref[...] @ w_ref[...] + b_ref[...]


def _cls_matmul(h, w, b):
    Np = h.shape[0]
    return pl.pallas_call(
        _cls_kernel,
        out_shape=jax.ShapeDtypeStruct((Np, w.shape[1]), jnp.float32),
    )(h, w, b[None, :])


def _pna_conv(x, src, dst, Wpre, bpre, Wpost, bpost, Wlin, blin, deg, avg_log):
    h = jnp.concatenate([x[dst], x[src]], axis=-1) @ Wpre + bpre
    degc = jnp.maximum(deg, 1.0)
    mean = jax.ops.segment_sum(h, dst, num_segments=N) / degc[:, None]
    mean2 = jax.ops.segment_sum(h * h, dst, num_segments=N) / degc[:, None]
    std = jnp.sqrt(jax.nn.relu(mean2 - mean * mean) + 1e-5)
    mx = jax.ops.segment_max(h, dst, num_segments=N)
    mx = jnp.where(jnp.isfinite(mx), mx, 0.0)
    mn = -jax.ops.segment_max(-h, dst, num_segments=N)
    mn = jnp.where(jnp.isfinite(mn), mn, 0.0)
    aggs = jnp.concatenate([mean, mn, mx, std], axis=-1)
    amp = (jnp.log(degc + 1.0) / avg_log)[:, None]
    att = (avg_log / jnp.log(degc + 1.0))[:, None]
    scaled = jnp.concatenate([aggs, aggs * amp, aggs * att], axis=-1)
    out = jnp.concatenate([x, scaled], axis=-1) @ Wpost + bpost
    return out @ Wlin + blin


def _bn(h, g, b):
    m = jnp.mean(h, axis=0)
    v = jnp.var(h, axis=0)
    return (h - m) / jnp.sqrt(v + 1e-5) * g + b


def kernel(x, edge_index,
           pre_w1, pre_b1, post_w1, post_b1, lin_w1, lin_b1, bn_g1, bn_b1,
           pre_w2, pre_b2, post_w2, post_b2, lin_w2, lin_b2, bn_g2, bn_b2,
           pre_w3, pre_b3, post_w3, post_b3, lin_w3, lin_b3, bn_g3, bn_b3,
           pre_w4, pre_b4, post_w4, post_b4, lin_w4, lin_b4, bn_g4, bn_b4,
           cls_w, cls_b):
    p = dict(locals())
    src = edge_index[0]
    dst = edge_index[1]
    deg = jax.ops.segment_sum(jnp.ones((E,), jnp.float32), dst, num_segments=N)
    avg_log = jnp.mean(jnp.log(deg + 1.0))
    h = x
    for li in range(1, 5):
        h = _pna_conv(h, src, dst,
                      p[f"pre_w{li}"], p[f"pre_b{li}"],
                      p[f"post_w{li}"], p[f"post_b{li}"],
                      p[f"lin_w{li}"], p[f"lin_b{li}"],
                      deg, avg_log)
        h = _bn(jax.nn.relu(h), p[f"bn_g{li}"], p[f"bn_b{li}"])
    return _cls_matmul(h, p["cls_w"], p["cls_b"])


# speedup vs baseline: 1.8452x; 1.8452x over previous
import functools

import jax
import jax.numpy as jnp
from jax import lax
from jax.experimental import pallas as pl
from jax.experimental.pallas import tpu as pltpu
from jax.experimental.pallas import tpu_sc as plsc

N = 10000
E = 320000
NPT = 160
NR = 64
NPAD = NPT * NR
G = 128
EP = E + 2 * G
C = 128
DUMP = NPT


def _fused_body(h2, idxr, ldstr, startsr, s1o, s2o, mxo, mno,
                batch, idxv, ldstv, startsv,
                acc_s1, acc_s2, acc_mx, acc_mn, sem):
    cc = lax.axis_index("c")
    ss = lax.axis_index("s")
    w = cc * 16 + ss

    pltpu.sync_copy(startsr, startsv)

    neg = jnp.full((16,), -jnp.inf, dtype=jnp.float32)
    pos = jnp.full((16,), jnp.inf, dtype=jnp.float32)
    zero = jnp.zeros((16,), dtype=jnp.float32)
    iota16 = lax.iota(jnp.int32, 16)

    for rr in range(2):
        r = 2 * w + rr

        def initrow(i, carry):
            for j in range(C // 16):
                sl = pl.ds(j * 16, 16)
                acc_s1[i, sl] = zero
                acc_s2[i, sl] = zero
                acc_mx[i, sl] = neg
                acc_mn[i, sl] = pos
            return carry

        lax.fori_loop(0, NPT + 1, initrow, 0)

        sv = startsv[pl.ds(r, 16)]
        st = sv[0]
        en = sv[1]
        lo = (st // 8) * 8
        nb = (en - lo + G - 1) // G

        def batch_body(b, carry):
            base = lo + b * G
            pltpu.sync_copy(idxr.at[pl.ds(base, G)], idxv)
            pltpu.sync_copy(ldstr.at[pl.ds(base, G)], ldstv.at[pl.ds(0, G)])
            pltpu.async_copy(h2.at[idxv], batch, sem).wait()
            for i in range(G // 16):
                sl = pl.ds(i * 16, 16)
                l16 = ldstv[sl]
                posn = base + i * 16 + iota16
                ok = (posn >= st) & (posn < en)
                ldstv[sl] = jnp.where(ok, l16, DUMP)

            def rmw(e, carry2):
                le = ldstv[pl.ds(e, 16)][0]
                for j in range(C // 16):
                    sl = pl.ds(j * 16, 16)
                    v = batch[e, sl]
                    acc_s1[le, sl] = acc_s1[le, sl] + v
                    acc_s2[le, sl] = acc_s2[le, sl] + v * v
                    acc_mx[le, sl] = jnp.maximum(acc_mx[le, sl], v)
                    acc_mn[le, sl] = jnp.minimum(acc_mn[le, sl], v)
                return carry2

            lax.fori_loop(0, G, rmw, 0)
            return carry

        lax.fori_loop(0, nb, batch_body, 0)

        osl = pl.ds(r * NPT, NPT)
        tsl = pl.ds(0, NPT)
        pltpu.sync_copy(acc_s1.at[tsl], s1o.at[osl])
        pltpu.sync_copy(acc_s2.at[tsl], s2o.at[osl])
        pltpu.sync_copy(acc_mx.at[tsl], mxo.at[osl])
        pltpu.sync_copy(acc_mn.at[tsl], mno.at[osl])


_mesh = plsc.VectorSubcoreMesh(core_axis_name="c", subcore_axis_name="s")

_fused_reduce = functools.partial(
    pl.kernel,
    out_type=tuple(jax.ShapeDtypeStruct((NPAD, C), jnp.float32)
                   for _ in range(4)),
    mesh=_mesh,
    scratch_types=[
        pltpu.VMEM((G, C), jnp.float32),
        pltpu.VMEM((G,), jnp.int32),
        pltpu.VMEM((G + 16,), jnp.int32),
        pltpu.VMEM((80,), jnp.int32),
        pltpu.VMEM((NPT + 1, C), jnp.float32),
        pltpu.VMEM((NPT + 1, C), jnp.float32),
        pltpu.VMEM((NPT + 1, C), jnp.float32),
        pltpu.VMEM((NPT + 1, C), jnp.float32),
        pltpu.SemaphoreType.DMA,
    ],
)(_fused_body)


def _bn(h, g, b):
    m = jnp.mean(h, axis=0)
    v = jnp.var(h, axis=0)
    return (h - m) / jnp.sqrt(v + 1e-5) * g + b


def kernel(x, edge_index,
           pre_w1, pre_b1, post_w1, post_b1, lin_w1, lin_b1, bn_g1, bn_b1,
           pre_w2, pre_b2, post_w2, post_b2, lin_w2, lin_b2, bn_g2, bn_b2,
           pre_w3, pre_b3, post_w3, post_b3, lin_w3, lin_b3, bn_g3, bn_b3,
           pre_w4, pre_b4, post_w4, post_b4, lin_w4, lin_b4, bn_g4, bn_b4,
           cls_w, cls_b):
    p = dict(locals())
    src = edge_index[0]
    dst = edge_index[1]
    deg = jax.ops.segment_sum(jnp.ones((E,), jnp.float32), dst, num_segments=N)
    avg_log = jnp.mean(jnp.log(deg + 1.0))
    degc = jnp.maximum(deg, 1.0)
    amp = (jnp.log(degc + 1.0) / avg_log)[:, None]
    att = (avg_log / jnp.log(degc + 1.0))[:, None]

    perm = jnp.argsort(dst, stable=True).astype(jnp.int32)
    dst_s = dst[perm]
    ldst = (dst_s % NPT).astype(jnp.int32)
    bounds = NPT * jnp.arange(NR + 1, dtype=jnp.int32)
    starts = jnp.searchsorted(dst_s, bounds, side="left").astype(jnp.int32)
    starts_p = jnp.concatenate([starts, jnp.zeros((80 - NR - 1,), jnp.int32)])
    padi = jnp.zeros((EP - E,), jnp.int32)
    ldst_p = jnp.concatenate([ldst, padi])

    h_cur = x
    for li in range(1, 5):
        Wpre, bpre = p[f"pre_w{li}"], p[f"pre_b{li}"]
        Wpost, bpost = p[f"post_w{li}"], p[f"post_b{li}"]
        Wlin, blin = p[f"lin_w{li}"], p[f"lin_b{li}"]
        fin = Wpre.shape[1]
        h = jnp.concatenate([h_cur[dst], h_cur[src]], axis=-1) @ Wpre + bpre
        npass = fin // C
        h2 = jnp.reshape(h, (E * npass, C))
        parts = []
        for k in range(npass):
            idxk = jnp.concatenate([npass * perm + k, padi])
            parts.append(_fused_reduce(h2, idxk, ldst_p, starts_p))
        S1 = jnp.concatenate([parts[k][0][:N] for k in range(npass)], axis=1)
        S2 = jnp.concatenate([parts[k][1][:N] for k in range(npass)], axis=1)
        Mxr = jnp.concatenate([parts[k][2][:N] for k in range(npass)], axis=1)
        Mnr = jnp.concatenate([parts[k][3][:N] for k in range(npass)], axis=1)
        mean = S1 / degc[:, None]
        mean2 = S2 / degc[:, None]
        std = jnp.sqrt(jax.nn.relu(mean2 - mean * mean) + 1e-5)
        mx = jnp.where(jnp.isfinite(Mxr), Mxr, 0.0)
        mn = jnp.where(jnp.isfinite(Mnr), Mnr, 0.0)
        aggs = jnp.concatenate([mean, mn, mx, std], axis=-1)
        scaled = jnp.concatenate([aggs, aggs * amp, aggs * att], axis=-1)
        out = jnp.concatenate([h_cur, scaled], axis=-1) @ Wpost + bpost
        out = out @ Wlin + blin
        h_cur = _bn(jax.nn.relu(out), p[f"bn_g{li}"], p[f"bn_b{li}"])
    return h_cur @ p["cls_w"] + p["cls_b"]
